# baseline (speedup 1.0000x reference)
import jax
import jax.numpy as jnp
from jax import lax
from jax.experimental import pallas as pl
from jax.experimental.pallas import tpu as pltpu
from jax.experimental.pallas import tpu_sc as plsc

_N = 10000
_NPAD = 10240
_D = 128
_E = 320000
_NC = 2
_NS = 16
_NW = _NC * _NS
_CH = 128
_G = 80
_EPAD = _NW * _G * _CH
_RPT = _NPAD // _NS
_ZR = 128
_NDUMP = _RPT // _ZR
_BLK = 256


def _sc_agg_body(with_counts, *refs):
    if with_counts:
        (table, src, dst, agg_out, cnt_out,
         srcw0, srcw1, dstw0, dstw1, buf0, cnt_v, agg_sh,
         sem_g0, sem_s0, sem_i0, sem_i1) = refs
    else:
        (table, src, dst, agg_out,
         srcw0, srcw1, dstw0, dstw1, buf0, agg_sh,
         sem_g0, sem_s0, sem_i0, sem_i1) = refs
    c = lax.axis_index("c")
    s = lax.axis_index("s")
    wid = s * _NC + c
    ebase = wid * _G * _CH
    zeros16 = jnp.zeros((16,), jnp.float32)
    ones16 = jnp.ones((16,), jnp.float32)

    def _zb(i, carry):
        r = i // (_D // 16)
        col = (i % (_D // 16)) * 16
        buf0[r, pl.ds(col, 16)] = zeros16
        return carry
    lax.fori_loop(0, _ZR * _D // 16, _zb, 0)
    base = s * _RPT
    for r in range(_NDUMP):
        pltpu.sync_copy(buf0, agg_sh.at[pl.ds(base + r * _ZR, _ZR)])
    if with_counts:
        def _zc(i, carry):
            cnt_v[pl.ds(i * 16, 16)] = zeros16
            return carry
        lax.fori_loop(0, _NPAD // 16, _zc, 0)
    plsc.subcore_barrier()

    def _fire_idx(j, edges, w, sem):
        jc = jnp.minimum(j, _G - 1)
        pltpu.async_copy(edges.at[pl.ds(ebase + jc * _CH, _CH)], w, sem)

    def _drain_idx(w, sem):
        pltpu.make_async_copy(src.at[pl.ds(0, _CH)], w, sem).wait()

    def _counts(w):
        if with_counts:
            for k in range(_CH // 16):
                idx = w[pl.ds(k * 16, 16)]
                plsc.addupdate_scatter(cnt_v, [idx], ones16)

    _fire_idx(0, src, srcw0, sem_i0)
    _fire_idx(0, dst, dstw0, sem_i0)
    _fire_idx(1, src, srcw1, sem_i1)
    _fire_idx(1, dst, dstw1, sem_i1)
    _drain_idx(srcw0, sem_i0)
    _drain_idx(dstw0, sem_i0)

    def _step(it, carry):
        j0 = it * 2
        pltpu.async_copy(table.at[srcw0], buf0, sem_g0).wait()
        pltpu.sync_copy(buf0, agg_sh.at[dstw0], add=True)
        _counts(dstw0)
        _fire_idx(j0 + 2, src, srcw0, sem_i0)
        _fire_idx(j0 + 2, dst, dstw0, sem_i0)
        _drain_idx(srcw1, sem_i1)
        _drain_idx(dstw1, sem_i1)
        pltpu.async_copy(table.at[srcw1], buf0, sem_g0).wait()
        pltpu.sync_copy(buf0, agg_sh.at[dstw1], add=True)
        _counts(dstw1)
        _fire_idx(j0 + 3, src, srcw1, sem_i1)
        _fire_idx(j0 + 3, dst, dstw1, sem_i1)
        _drain_idx(srcw0, sem_i0)
        _drain_idx(dstw0, sem_i0)
        return carry
    lax.fori_loop(0, _G // 2, _step, 0)
    _drain_idx(srcw1, sem_i1)
    _drain_idx(dstw1, sem_i1)
    plsc.subcore_barrier()

    for r in range(_NDUMP):
        pltpu.sync_copy(agg_sh.at[pl.ds(base + r * _ZR, _ZR)], buf0)
        pltpu.sync_copy(buf0, agg_out.at[c, pl.ds(base + r * _ZR, _ZR)])
    if with_counts:
        pltpu.sync_copy(cnt_v, cnt_out.at[c, s])


def _make_sc_agg(with_counts):
    mesh = plsc.VectorSubcoreMesh(core_axis_name="c", subcore_axis_name="s",
                                  num_cores=_NC, num_subcores=_NS)
    out_type = [jax.ShapeDtypeStruct((_NC, _NPAD, _D), jnp.float32)]
    if with_counts:
        out_type.append(jax.ShapeDtypeStruct((_NC, _NS, _NPAD), jnp.float32))
    scratch = [
        pltpu.VMEM((_CH,), jnp.int32),
        pltpu.VMEM((_CH,), jnp.int32),
        pltpu.VMEM((_CH,), jnp.int32),
        pltpu.VMEM((_CH,), jnp.int32),
        pltpu.VMEM((_CH, _D), jnp.float32),
    ]
    if with_counts:
        scratch.append(pltpu.VMEM((_NPAD,), jnp.float32))
    scratch.append(pltpu.VMEM_SHARED((_NPAD, _D), jnp.float32))
    scratch.extend([pltpu.SemaphoreType.DMA] * 4)

    def body(*refs):
        _sc_agg_body(with_counts, *refs)
    return pl.kernel(body, out_type=tuple(out_type), mesh=mesh,
                     compiler_params=pltpu.CompilerParams(needs_layout_passes=False),
                     scratch_types=tuple(scratch))


_SC_CACHE = {}


def _sc_agg(with_counts, *args):
    if with_counts not in _SC_CACHE:
        _SC_CACHE[with_counts] = _make_sc_agg(with_counts)
    return _SC_CACHE[with_counts](*args)


def _tc_layer1_body(agg_ref, cnt_ref, x_ref, wl_ref, wr_ref, b_ref, out_ref):
    agg = agg_ref[0] + agg_ref[1]
    cnt = jnp.sum(cnt_ref[...].reshape(_NC * _NS, _BLK), axis=0)
    inv = 1.0 / jnp.maximum(cnt, 1.0)
    mean = agg * inv[:, None]
    h = lax.dot_general(mean, wl_ref[...], (((1,), (1,)), ((), ())),
                        preferred_element_type=jnp.float32)
    h = h + lax.dot_general(x_ref[...], wr_ref[...], (((1,), (1,)), ((), ())),
                            preferred_element_type=jnp.float32)
    h = h + b_ref[...]
    out_ref[...] = jnp.maximum(h, 0.0)


def _tc_layer2_body(agg_ref, cnt_ref, x_ref, wl_ref, wr_ref, b_ref,
                    wo_ref, bo_ref, h_ref, logit_ref):
    agg = agg_ref[0] + agg_ref[1]
    cnt = jnp.sum(cnt_ref[...].reshape(_NC * _NS, _BLK), axis=0)
    inv = 1.0 / jnp.maximum(cnt, 1.0)
    mean = agg * inv[:, None]
    h = lax.dot_general(mean, wl_ref[...], (((1,), (1,)), ((), ())),
                        preferred_element_type=jnp.float32)
    h = h + lax.dot_general(x_ref[...], wr_ref[...], (((1,), (1,)), ((), ())),
                            preferred_element_type=jnp.float32)
    h = h + b_ref[...]
    h = jnp.maximum(h, 0.0)
    h_ref[...] = h
    logit_ref[...] = lax.dot_general(h, wo_ref[...], (((1,), (1,)), ((), ())),
                                     preferred_element_type=jnp.float32) + bo_ref[...]


_agg_spec = pl.BlockSpec((_NC, _BLK, _D), lambda i: (0, i, 0))
_cnt_spec = pl.BlockSpec((_NC, _NS, _BLK), lambda i: (0, 0, i))
_row_spec = pl.BlockSpec((_BLK, _D), lambda i: (i, 0))
_w_spec = pl.BlockSpec((_D, _D), lambda i: (0, 0))
_b_spec = pl.BlockSpec((1, _D), lambda i: (0, 0))


def _tc_layer1(agg, cnt, x, wl, wr, b):
    return pl.pallas_call(
        _tc_layer1_body,
        grid=(_NPAD // _BLK,),
        in_specs=[_agg_spec, _cnt_spec, _row_spec, _w_spec, _w_spec, _b_spec],
        out_specs=_row_spec,
        out_shape=jax.ShapeDtypeStruct((_NPAD, _D), jnp.float32),
    )(agg, cnt, x, wl, wr, b)


def _tc_layer2(agg, cnt, h1, wl, wr, b, wo, bo):
    return pl.pallas_call(
        _tc_layer2_body,
        grid=(_NPAD // _BLK,),
        in_specs=[_agg_spec, _cnt_spec, _row_spec, _w_spec, _w_spec, _b_spec,
                  _w_spec, _b_spec],
        out_specs=(_row_spec, _row_spec),
        out_shape=(jax.ShapeDtypeStruct((_NPAD, _D), jnp.float32),
                   jax.ShapeDtypeStruct((_NPAD, _D), jnp.float32)),
    )(agg, cnt, h1, wl, wr, b, wo, bo)


def kernel(x, edge_index, W1l, b1l, W1r, W2l, b2l, W2r, Wo, bo):
    src = edge_index[0]
    dst = edge_index[1]
    pad = jnp.full((_EPAD - _E,), _N, dtype=jnp.int32)
    srcp = jnp.concatenate([src, pad])
    dstp = jnp.concatenate([dst, pad])
    xp = jnp.zeros((_NPAD, _D), jnp.float32).at[:_N].set(x)
    agg1, cnt = _sc_agg(True, xp, srcp, dstp)
    h1 = _tc_layer1(agg1, cnt, xp, W1l, W1r, b1l.reshape(1, _D))
    (agg2,) = _sc_agg(False, h1, srcp, dstp)
    wo_pad = jnp.zeros((_D, _D), jnp.float32).at[:Wo.shape[0]].set(Wo)
    bo_pad = jnp.zeros((1, _D), jnp.float32).at[0, :bo.shape[0]].set(bo)
    h2, logits_pad = _tc_layer2(agg2, cnt, h1, W2l, W2r, b2l.reshape(1, _D),
                                wo_pad, bo_pad)
    return (logits_pad[:_N, :Wo.shape[0]], h2[:_N])


# speedup vs baseline: 1.6648x; 1.6648x over previous
import jax
import jax.numpy as jnp
from jax import lax
from jax.experimental import pallas as pl
from jax.experimental.pallas import tpu as pltpu
from jax.experimental.pallas import tpu_sc as plsc

_N = 10000
_NPAD = 10240
_D = 128
_E = 320000
_NC = 2
_NS = 16
_NW = _NC * _NS
_CH = 80
_G = 125
_RPT = _NPAD // _NS
_ZR = 128
_NDUMP = _RPT // _ZR
_BLK = 256


def _sc_agg_body(with_counts, *refs):
    if with_counts:
        (table, src, dst, agg_out, cnt_out,
         srcw0, srcw1, dstw0, dstw1, buf0, buf1, bounce, cnt_v, agg_sh,
         sem_g0, sem_s0, sem_i0, sem_i1) = refs
    else:
        (table, src, dst, agg_out,
         srcw0, srcw1, dstw0, dstw1, buf0, buf1, bounce, agg_sh,
         sem_g0, sem_s0, sem_i0, sem_i1) = refs
    c = lax.axis_index("c")
    s = lax.axis_index("s")
    wid = s * _NC + c
    ebase = wid * _G * _CH
    zeros16 = jnp.zeros((16,), jnp.float32)
    ones16 = jnp.ones((16,), jnp.float32)

    def _zb(i, carry):
        r = i // (_D // 16)
        col = (i % (_D // 16)) * 16
        bounce[r, pl.ds(col, 16)] = zeros16
        return carry
    lax.fori_loop(0, _ZR * _D // 16, _zb, 0)
    base = s * _RPT
    for r in range(_NDUMP):
        pltpu.sync_copy(bounce, agg_sh.at[pl.ds(base + r * _ZR, _ZR)])
    if with_counts:
        def _zc(i, carry):
            cnt_v[pl.ds(i * 16, 16)] = zeros16
            return carry
        lax.fori_loop(0, _NPAD // 16, _zc, 0)
    plsc.subcore_barrier()

    def _counts(w):
        if with_counts:
            for k in range(_CH // 16):
                idx = w[pl.ds(k * 16, 16)]
                plsc.addupdate_scatter(cnt_v, [idx], ones16)

    def _load_idx(j, sw, dw):
        b = ebase + j * _CH
        pltpu.sync_copy(src.at[pl.ds(b, _CH)], sw)
        pltpu.sync_copy(dst.at[pl.ds(b, _CH)], dw)

    def _drain_scatter(buf):
        pltpu.make_async_copy(buf, agg_sh.at[pl.ds(0, _CH)], sem_s0).wait()

    _load_idx(0, srcw0, dstw0)
    pltpu.async_copy(table.at[srcw0], buf0, sem_g0).wait()
    pltpu.async_copy(buf0, agg_sh.at[dstw0], sem_s0, add=True)
    _counts(dstw0)

    def _step(it, carry):
        j0 = 1 + it * 2
        _load_idx(j0, srcw1, dstw1)
        pltpu.async_copy(table.at[srcw1], buf1, sem_g0).wait()
        _drain_scatter(buf0)
        pltpu.async_copy(buf1, agg_sh.at[dstw1], sem_s0, add=True)
        _counts(dstw1)
        _load_idx(j0 + 1, srcw0, dstw0)
        pltpu.async_copy(table.at[srcw0], buf0, sem_g0).wait()
        _drain_scatter(buf1)
        pltpu.async_copy(buf0, agg_sh.at[dstw0], sem_s0, add=True)
        _counts(dstw0)
        return carry
    lax.fori_loop(0, (_G - 1) // 2, _step, 0)
    _drain_scatter(buf0)
    plsc.subcore_barrier()

    for r in range(_NDUMP):
        pltpu.sync_copy(agg_sh.at[pl.ds(base + r * _ZR, _ZR)], bounce)
        pltpu.sync_copy(bounce, agg_out.at[c, pl.ds(base + r * _ZR, _ZR)])
    if with_counts:
        pltpu.sync_copy(cnt_v, cnt_out.at[c, s])


def _make_sc_agg(with_counts):
    mesh = plsc.VectorSubcoreMesh(core_axis_name="c", subcore_axis_name="s",
                                  num_cores=_NC, num_subcores=_NS)
    out_type = [jax.ShapeDtypeStruct((_NC, _NPAD, _D), jnp.float32)]
    if with_counts:
        out_type.append(jax.ShapeDtypeStruct((_NC, _NS, _NPAD), jnp.float32))
    scratch = [
        pltpu.VMEM((_CH,), jnp.int32),
        pltpu.VMEM((_CH,), jnp.int32),
        pltpu.VMEM((_CH,), jnp.int32),
        pltpu.VMEM((_CH,), jnp.int32),
        pltpu.VMEM((_CH, _D), jnp.float32),
        pltpu.VMEM((_CH, _D), jnp.float32),
        pltpu.VMEM((_ZR, _D), jnp.float32),
    ]
    if with_counts:
        scratch.append(pltpu.VMEM((_NPAD,), jnp.float32))
    scratch.append(pltpu.VMEM_SHARED((_NPAD, _D), jnp.float32))
    scratch.extend([pltpu.SemaphoreType.DMA] * 4)

    def body(*refs):
        _sc_agg_body(with_counts, *refs)
    return pl.kernel(body, out_type=tuple(out_type), mesh=mesh,
                     compiler_params=pltpu.CompilerParams(needs_layout_passes=False),
                     scratch_types=tuple(scratch))


_SC_CACHE = {}


def _sc_agg(with_counts, *args):
    if with_counts not in _SC_CACHE:
        _SC_CACHE[with_counts] = _make_sc_agg(with_counts)
    return _SC_CACHE[with_counts](*args)


def _tc_layer1_body(agg_ref, cnt_ref, x_ref, wl_ref, wr_ref, b_ref, out_ref):
    agg = agg_ref[0] + agg_ref[1]
    cnt = jnp.sum(cnt_ref[...].reshape(_NC * _NS, _BLK), axis=0)
    inv = 1.0 / jnp.maximum(cnt, 1.0)
    mean = agg * inv[:, None]
    h = lax.dot_general(mean, wl_ref[...], (((1,), (1,)), ((), ())),
                        preferred_element_type=jnp.float32)
    h = h + lax.dot_general(x_ref[...], wr_ref[...], (((1,), (1,)), ((), ())),
                            preferred_element_type=jnp.float32)
    h = h + b_ref[...]
    out_ref[...] = jnp.maximum(h, 0.0)


def _tc_layer2_body(agg_ref, cnt_ref, x_ref, wl_ref, wr_ref, b_ref,
                    wo_ref, bo_ref, h_ref, logit_ref):
    agg = agg_ref[0] + agg_ref[1]
    cnt = jnp.sum(cnt_ref[...].reshape(_NC * _NS, _BLK), axis=0)
    inv = 1.0 / jnp.maximum(cnt, 1.0)
    mean = agg * inv[:, None]
    h = lax.dot_general(mean, wl_ref[...], (((1,), (1,)), ((), ())),
                        preferred_element_type=jnp.float32)
    h = h + lax.dot_general(x_ref[...], wr_ref[...], (((1,), (1,)), ((), ())),
                            preferred_element_type=jnp.float32)
    h = h + b_ref[...]
    h = jnp.maximum(h, 0.0)
    h_ref[...] = h
    logit_ref[...] = lax.dot_general(h, wo_ref[...], (((1,), (1,)), ((), ())),
                                     preferred_element_type=jnp.float32) + bo_ref[...]


_agg_spec = pl.BlockSpec((_NC, _BLK, _D), lambda i: (0, i, 0))
_cnt_spec = pl.BlockSpec((_NC, _NS, _BLK), lambda i: (0, 0, i))
_row_spec = pl.BlockSpec((_BLK, _D), lambda i: (i, 0))
_w_spec = pl.BlockSpec((_D, _D), lambda i: (0, 0))
_b_spec = pl.BlockSpec((1, _D), lambda i: (0, 0))


def _tc_layer1(agg, cnt, x, wl, wr, b):
    return pl.pallas_call(
        _tc_layer1_body,
        grid=(_NPAD // _BLK,),
        in_specs=[_agg_spec, _cnt_spec, _row_spec, _w_spec, _w_spec, _b_spec],
        out_specs=_row_spec,
        out_shape=jax.ShapeDtypeStruct((_NPAD, _D), jnp.float32),
    )(agg, cnt, x, wl, wr, b)


def _tc_layer2(agg, cnt, h1, wl, wr, b, wo, bo):
    return pl.pallas_call(
        _tc_layer2_body,
        grid=(_NPAD // _BLK,),
        in_specs=[_agg_spec, _cnt_spec, _row_spec, _w_spec, _w_spec, _b_spec,
                  _w_spec, _b_spec],
        out_specs=(_row_spec, _row_spec),
        out_shape=(jax.ShapeDtypeStruct((_NPAD, _D), jnp.float32),
                   jax.ShapeDtypeStruct((_NPAD, _D), jnp.float32)),
    )(agg, cnt, h1, wl, wr, b, wo, bo)


def kernel(x, edge_index, W1l, b1l, W1r, W2l, b2l, W2r, Wo, bo):
    src = edge_index[0]
    dst = edge_index[1]
    xp = jnp.zeros((_NPAD, _D), jnp.float32).at[:_N].set(x)
    agg1, cnt = _sc_agg(True, xp, src, dst)
    h1 = _tc_layer1(agg1, cnt, xp, W1l, W1r, b1l.reshape(1, _D))
    (agg2,) = _sc_agg(False, h1, src, dst)
    wo_pad = jnp.zeros((_D, _D), jnp.float32).at[:Wo.shape[0]].set(Wo)
    bo_pad = jnp.zeros((1, _D), jnp.float32).at[0, :bo.shape[0]].set(bo)
    h2, logits_pad = _tc_layer2(agg2, cnt, h1, W2l, W2r, b2l.reshape(1, _D),
                                wo_pad, bo_pad)
    return (logits_pad[:_N, :Wo.shape[0]], h2[:_N])


# speedup vs baseline: 1.8679x; 1.1220x over previous
import jax
import jax.numpy as jnp
from jax import lax
from jax.experimental import pallas as pl
from jax.experimental.pallas import tpu as pltpu
from jax.experimental.pallas import tpu_sc as plsc

_N = 10000
_NPAD = 10240
_D = 128
_E = 320000
_NC = 2
_NS = 16
_NW = _NC * _NS
_CH = 80
_G = 125
_RPT = _NPAD // _NS
_ZR = 128
_NDUMP = _RPT // _ZR
_BLK = 256


def _sc_agg_body(with_counts, *refs):
    if with_counts:
        (table, il, agg_out, cnt_out,
         idxw0, idxw1, dstw0, dstw1, buf0, buf1, bounce, cnt_v, agg_sh,
         sem_g0, sem_s0, sem_i0, sem_i1) = refs
    else:
        (table, il, agg_out,
         idxw0, idxw1, dstw0, dstw1, buf0, buf1, bounce, agg_sh,
         sem_g0, sem_s0, sem_i0, sem_i1) = refs
    c = lax.axis_index("c")
    s = lax.axis_index("s")
    wid = s * _NC + c
    ebase = wid * _G * 2 * _CH
    zeros16 = jnp.zeros((16,), jnp.float32)
    ones16 = jnp.ones((16,), jnp.float32)

    def _zb(i, carry):
        r = i // (_D // 16)
        col = (i % (_D // 16)) * 16
        bounce[r, pl.ds(col, 16)] = zeros16
        return carry
    lax.fori_loop(0, _ZR * _D // 16, _zb, 0)
    base = s * _RPT
    for r in range(_NDUMP):
        pltpu.sync_copy(bounce, agg_sh.at[pl.ds(base + r * _ZR, _ZR)])
    if with_counts:
        def _zc(i, carry):
            cnt_v[pl.ds(i * 16, 16)] = zeros16
            return carry
        lax.fori_loop(0, _NPAD // 16, _zc, 0)
    plsc.subcore_barrier()

    def _counts(w):
        if with_counts:
            for k in range(_CH // 16):
                idx = w[pl.ds(k * 16, 16)]
                plsc.addupdate_scatter(cnt_v, [idx], ones16)

    def _load_idx(j, iw, dw):
        b = ebase + j * 2 * _CH
        pltpu.sync_copy(il.at[pl.ds(b, 2 * _CH)], iw)
        for k in range(_CH // 16):
            dw[pl.ds(k * 16, 16)] = iw[pl.ds(_CH + k * 16, 16)]

    def _drain_scatter(buf):
        pltpu.make_async_copy(buf, agg_sh.at[pl.ds(0, _CH)], sem_s0).wait()

    _load_idx(0, idxw0, dstw0)
    pltpu.async_copy(table.at[idxw0.at[pl.ds(0, _CH)]], buf0, sem_g0).wait()
    pltpu.async_copy(buf0, agg_sh.at[dstw0], sem_s0, add=True)
    _counts(dstw0)

    def _step(it, carry):
        j0 = 1 + it * 2
        _load_idx(j0, idxw1, dstw1)
        pltpu.async_copy(table.at[idxw1.at[pl.ds(0, _CH)]], buf1, sem_g0).wait()
        _drain_scatter(buf0)
        pltpu.async_copy(buf1, agg_sh.at[dstw1], sem_s0, add=True)
        _counts(dstw1)
        _load_idx(j0 + 1, idxw0, dstw0)
        pltpu.async_copy(table.at[idxw0.at[pl.ds(0, _CH)]], buf0, sem_g0).wait()
        _drain_scatter(buf1)
        pltpu.async_copy(buf0, agg_sh.at[dstw0], sem_s0, add=True)
        _counts(dstw0)
        return carry
    lax.fori_loop(0, (_G - 1) // 2, _step, 0)
    _drain_scatter(buf0)
    plsc.subcore_barrier()

    for r in range(_NDUMP):
        pltpu.sync_copy(agg_sh.at[pl.ds(base + r * _ZR, _ZR)], bounce)
        pltpu.sync_copy(bounce, agg_out.at[c, pl.ds(base + r * _ZR, _ZR)])
    if with_counts:
        pltpu.sync_copy(cnt_v, cnt_out.at[c, s])


def _make_sc_agg(with_counts):
    mesh = plsc.VectorSubcoreMesh(core_axis_name="c", subcore_axis_name="s",
                                  num_cores=_NC, num_subcores=_NS)
    out_type = [jax.ShapeDtypeStruct((_NC, _NPAD, _D), jnp.float32)]
    if with_counts:
        out_type.append(jax.ShapeDtypeStruct((_NC, _NS, _NPAD), jnp.float32))
    scratch = [
        pltpu.VMEM((2 * _CH,), jnp.int32),
        pltpu.VMEM((2 * _CH,), jnp.int32),
        pltpu.VMEM((_CH,), jnp.int32),
        pltpu.VMEM((_CH,), jnp.int32),
        pltpu.VMEM((_CH, _D), jnp.float32),
        pltpu.VMEM((_CH, _D), jnp.float32),
        pltpu.VMEM((_ZR, _D), jnp.float32),
    ]
    if with_counts:
        scratch.append(pltpu.VMEM((_NPAD,), jnp.float32))
    scratch.append(pltpu.VMEM_SHARED((_NPAD, _D), jnp.float32))
    scratch.extend([pltpu.SemaphoreType.DMA] * 4)

    def body(*refs):
        _sc_agg_body(with_counts, *refs)
    return pl.kernel(body, out_type=tuple(out_type), mesh=mesh,
                     compiler_params=pltpu.CompilerParams(needs_layout_passes=False),
                     scratch_types=tuple(scratch))


_SC_CACHE = {}


def _sc_agg(with_counts, *args):
    if with_counts not in _SC_CACHE:
        _SC_CACHE[with_counts] = _make_sc_agg(with_counts)
    return _SC_CACHE[with_counts](*args)


def _tc_layer1_body(agg_ref, cnt_ref, x_ref, wl_ref, wr_ref, b_ref, out_ref):
    agg = agg_ref[0] + agg_ref[1]
    cnt = jnp.sum(cnt_ref[...].reshape(_NC * _NS, _BLK), axis=0)
    inv = 1.0 / jnp.maximum(cnt, 1.0)
    mean = agg * inv[:, None]
    h = lax.dot_general(mean, wl_ref[...], (((1,), (1,)), ((), ())),
                        preferred_element_type=jnp.float32)
    h = h + lax.dot_general(x_ref[...], wr_ref[...], (((1,), (1,)), ((), ())),
                            preferred_element_type=jnp.float32)
    h = h + b_ref[...]
    out_ref[...] = jnp.maximum(h, 0.0)


def _tc_layer2_body(agg_ref, cnt_ref, x_ref, wl_ref, wr_ref, b_ref,
                    wo_ref, bo_ref, h_ref, logit_ref):
    agg = agg_ref[0] + agg_ref[1]
    cnt = jnp.sum(cnt_ref[...].reshape(_NC * _NS, _BLK), axis=0)
    inv = 1.0 / jnp.maximum(cnt, 1.0)
    mean = agg * inv[:, None]
    h = lax.dot_general(mean, wl_ref[...], (((1,), (1,)), ((), ())),
                        preferred_element_type=jnp.float32)
    h = h + lax.dot_general(x_ref[...], wr_ref[...], (((1,), (1,)), ((), ())),
                            preferred_element_type=jnp.float32)
    h = h + b_ref[...]
    h = jnp.maximum(h, 0.0)
    h_ref[...] = h
    logit_ref[...] = lax.dot_general(h, wo_ref[...], (((1,), (1,)), ((), ())),
                                     preferred_element_type=jnp.float32) + bo_ref[...]


_agg_spec = pl.BlockSpec((_NC, _BLK, _D), lambda i: (0, i, 0))
_cnt_spec = pl.BlockSpec((_NC, _NS, _BLK), lambda i: (0, 0, i))
_row_spec = pl.BlockSpec((_BLK, _D), lambda i: (i, 0))
_w_spec = pl.BlockSpec((_D, _D), lambda i: (0, 0))
_b_spec = pl.BlockSpec((1, _D), lambda i: (0, 0))


def _tc_layer1(agg, cnt, x, wl, wr, b):
    return pl.pallas_call(
        _tc_layer1_body,
        grid=(_NPAD // _BLK,),
        in_specs=[_agg_spec, _cnt_spec, _row_spec, _w_spec, _w_spec, _b_spec],
        out_specs=_row_spec,
        out_shape=jax.ShapeDtypeStruct((_NPAD, _D), jnp.float32),
    )(agg, cnt, x, wl, wr, b)


def _tc_layer2(agg, cnt, h1, wl, wr, b, wo, bo):
    return pl.pallas_call(
        _tc_layer2_body,
        grid=(_NPAD // _BLK,),
        in_specs=[_agg_spec, _cnt_spec, _row_spec, _w_spec, _w_spec, _b_spec,
                  _w_spec, _b_spec],
        out_specs=(_row_spec, _row_spec),
        out_shape=(jax.ShapeDtypeStruct((_NPAD, _D), jnp.float32),
                   jax.ShapeDtypeStruct((_NPAD, _D), jnp.float32)),
    )(agg, cnt, h1, wl, wr, b, wo, bo)


def kernel(x, edge_index, W1l, b1l, W1r, W2l, b2l, W2r, Wo, bo):
    src = edge_index[0]
    dst = edge_index[1]
    il = jnp.stack([src.reshape(-1, _CH), dst.reshape(-1, _CH)],
                   axis=1).reshape(-1)
    xp = jnp.zeros((_NPAD, _D), jnp.float32).at[:_N].set(x)
    agg1, cnt = _sc_agg(True, xp, il)
    h1 = _tc_layer1(agg1, cnt, xp, W1l, W1r, b1l.reshape(1, _D))
    (agg2,) = _sc_agg(False, h1, il)
    wo_pad = jnp.zeros((_D, _D), jnp.float32).at[:Wo.shape[0]].set(Wo)
    bo_pad = jnp.zeros((1, _D), jnp.float32).at[0, :bo.shape[0]].set(bo)
    h2, logits_pad = _tc_layer2(agg2, cnt, h1, W2l, W2r, b2l.reshape(1, _D),
                                wo_pad, bo_pad)
    return (logits_pad[:_N, :Wo.shape[0]], h2[:_N])


# speedup vs baseline: 1.8819x; 1.0074x over previous
import jax
import jax.numpy as jnp
from jax import lax
from jax.experimental import pallas as pl
from jax.experimental.pallas import tpu as pltpu
from jax.experimental.pallas import tpu_sc as plsc

_N = 10000
_NPAD = 10240
_D = 128
_E = 320000
_NC = 2
_NS = 16
_NW = _NC * _NS
_CH = 80
_G = 125
_RPT = _NPAD // _NS
_ZR = 128
_NDUMP = _RPT // _ZR
_BLK = 256


def _sc_agg_body(with_counts, *refs):
    if with_counts:
        (table, il, agg_out, cnt_out,
         idxw0, idxw1, dstw0, dstw1, buf0, buf1, bounce, cnt_v, agg_sh,
         sem_g0, sem_s0, sem_i0, sem_i1) = refs
    else:
        (table, il, agg_out,
         idxw0, idxw1, dstw0, dstw1, buf0, buf1, bounce, agg_sh,
         sem_g0, sem_s0, sem_i0, sem_i1) = refs
    c = lax.axis_index("c")
    s = lax.axis_index("s")
    wid = s * _NC + c
    ebase = wid * _G * 2 * _CH
    zeros16 = jnp.zeros((16,), jnp.float32)
    ones16 = jnp.ones((16,), jnp.float32)

    def _zb(i, carry):
        r = i // (_D // 16)
        col = (i % (_D // 16)) * 16
        bounce[r, pl.ds(col, 16)] = zeros16
        return carry
    lax.fori_loop(0, _ZR * _D // 16, _zb, 0)
    base = s * _RPT
    for r in range(_NDUMP):
        pltpu.sync_copy(bounce, agg_sh.at[pl.ds(base + r * _ZR, _ZR)])
    if with_counts:
        def _zc(i, carry):
            cnt_v[pl.ds(i * 16, 16)] = zeros16
            return carry
        lax.fori_loop(0, _NPAD // 16, _zc, 0)
    plsc.subcore_barrier()

    def _counts(w):
        if with_counts:
            for k in range(_CH // 16):
                idx = w[pl.ds(k * 16, 16)]
                plsc.addupdate_scatter(cnt_v, [idx], ones16)

    def _load_idx(j, iw, dw):
        b = ebase + j * 2 * _CH
        pltpu.sync_copy(il.at[pl.ds(b, 2 * _CH)], iw)
        for k in range(_CH // 16):
            dw[pl.ds(k * 16, 16)] = iw[pl.ds(_CH + k * 16, 16)]

    def _fire_idx(j, iw, sem):
        jc = jnp.minimum(j, _G - 1)
        pltpu.async_copy(il.at[pl.ds(ebase + jc * 2 * _CH, 2 * _CH)], iw, sem)

    def _drain_idx(iw, dw, sem):
        pltpu.make_async_copy(il.at[pl.ds(0, 2 * _CH)], iw, sem).wait()
        for k in range(_CH // 16):
            dw[pl.ds(k * 16, 16)] = iw[pl.ds(_CH + k * 16, 16)]

    def _drain_scatter(buf):
        pltpu.make_async_copy(buf, agg_sh.at[pl.ds(0, _CH)], sem_s0).wait()

    _fire_idx(1, idxw1, sem_i1)
    _load_idx(0, idxw0, dstw0)
    pltpu.async_copy(table.at[idxw0.at[pl.ds(0, _CH)]], buf0, sem_g0).wait()
    pltpu.async_copy(buf0, agg_sh.at[dstw0], sem_s0, add=True)
    _counts(dstw0)

    def _step(it, carry):
        j0 = 1 + it * 2
        _drain_idx(idxw1, dstw1, sem_i1)
        pltpu.async_copy(table.at[idxw1.at[pl.ds(0, _CH)]], buf1, sem_g0).wait()
        _drain_scatter(buf0)
        _fire_idx(j0 + 1, idxw0, sem_i0)
        pltpu.async_copy(buf1, agg_sh.at[dstw1], sem_s0, add=True)
        _counts(dstw1)
        _drain_idx(idxw0, dstw0, sem_i0)
        pltpu.async_copy(table.at[idxw0.at[pl.ds(0, _CH)]], buf0, sem_g0).wait()
        _drain_scatter(buf1)
        _fire_idx(j0 + 2, idxw1, sem_i1)
        pltpu.async_copy(buf0, agg_sh.at[dstw0], sem_s0, add=True)
        _counts(dstw0)
        return carry
    lax.fori_loop(0, (_G - 1) // 2, _step, 0)
    _drain_scatter(buf0)
    pltpu.make_async_copy(il.at[pl.ds(0, 2 * _CH)], idxw1, sem_i1).wait()
    plsc.subcore_barrier()

    for r in range(_NDUMP):
        pltpu.sync_copy(agg_sh.at[pl.ds(base + r * _ZR, _ZR)], bounce)
        pltpu.sync_copy(bounce, agg_out.at[c, pl.ds(base + r * _ZR, _ZR)])
    if with_counts:
        pltpu.sync_copy(cnt_v, cnt_out.at[c, s])


def _make_sc_agg(with_counts):
    mesh = plsc.VectorSubcoreMesh(core_axis_name="c", subcore_axis_name="s",
                                  num_cores=_NC, num_subcores=_NS)
    out_type = [jax.ShapeDtypeStruct((_NC, _NPAD, _D), jnp.float32)]
    if with_counts:
        out_type.append(jax.ShapeDtypeStruct((_NC, _NS, _NPAD), jnp.float32))
    scratch = [
        pltpu.VMEM((2 * _CH,), jnp.int32),
        pltpu.VMEM((2 * _CH,), jnp.int32),
        pltpu.VMEM((_CH,), jnp.int32),
        pltpu.VMEM((_CH,), jnp.int32),
        pltpu.VMEM((_CH, _D), jnp.float32),
        pltpu.VMEM((_CH, _D), jnp.float32),
        pltpu.VMEM((_ZR, _D), jnp.float32),
    ]
    if with_counts:
        scratch.append(pltpu.VMEM((_NPAD,), jnp.float32))
    scratch.append(pltpu.VMEM_SHARED((_NPAD, _D), jnp.float32))
    scratch.extend([pltpu.SemaphoreType.DMA] * 4)

    def body(*refs):
        _sc_agg_body(with_counts, *refs)
    return pl.kernel(body, out_type=tuple(out_type), mesh=mesh,
                     compiler_params=pltpu.CompilerParams(needs_layout_passes=False),
                     scratch_types=tuple(scratch))


_SC_CACHE = {}


def _sc_agg(with_counts, *args):
    if with_counts not in _SC_CACHE:
        _SC_CACHE[with_counts] = _make_sc_agg(with_counts)
    return _SC_CACHE[with_counts](*args)


def _tc_layer1_body(agg_ref, cnt_ref, x_ref, wl_ref, wr_ref, b_ref, out_ref):
    agg = agg_ref[0] + agg_ref[1]
    cnt = jnp.sum(cnt_ref[...].reshape(_NC * _NS, _BLK), axis=0)
    inv = 1.0 / jnp.maximum(cnt, 1.0)
    mean = agg * inv[:, None]
    h = lax.dot_general(mean, wl_ref[...], (((1,), (1,)), ((), ())),
                        preferred_element_type=jnp.float32)
    h = h + lax.dot_general(x_ref[...], wr_ref[...], (((1,), (1,)), ((), ())),
                            preferred_element_type=jnp.float32)
    h = h + b_ref[...]
    out_ref[...] = jnp.maximum(h, 0.0)


def _tc_layer2_body(agg_ref, cnt_ref, x_ref, wl_ref, wr_ref, b_ref,
                    wo_ref, bo_ref, h_ref, logit_ref):
    agg = agg_ref[0] + agg_ref[1]
    cnt = jnp.sum(cnt_ref[...].reshape(_NC * _NS, _BLK), axis=0)
    inv = 1.0 / jnp.maximum(cnt, 1.0)
    mean = agg * inv[:, None]
    h = lax.dot_general(mean, wl_ref[...], (((1,), (1,)), ((), ())),
                        preferred_element_type=jnp.float32)
    h = h + lax.dot_general(x_ref[...], wr_ref[...], (((1,), (1,)), ((), ())),
                            preferred_element_type=jnp.float32)
    h = h + b_ref[...]
    h = jnp.maximum(h, 0.0)
    h_ref[...] = h
    logit_ref[...] = lax.dot_general(h, wo_ref[...], (((1,), (1,)), ((), ())),
                                     preferred_element_type=jnp.float32) + bo_ref[...]


_agg_spec = pl.BlockSpec((_NC, _BLK, _D), lambda i: (0, i, 0))
_cnt_spec = pl.BlockSpec((_NC, _NS, _BLK), lambda i: (0, 0, i))
_row_spec = pl.BlockSpec((_BLK, _D), lambda i: (i, 0))
_w_spec = pl.BlockSpec((_D, _D), lambda i: (0, 0))
_b_spec = pl.BlockSpec((1, _D), lambda i: (0, 0))


def _tc_layer1(agg, cnt, x, wl, wr, b):
    return pl.pallas_call(
        _tc_layer1_body,
        grid=(_NPAD // _BLK,),
        in_specs=[_agg_spec, _cnt_spec, _row_spec, _w_spec, _w_spec, _b_spec],
        out_specs=_row_spec,
        out_shape=jax.ShapeDtypeStruct((_NPAD, _D), jnp.float32),
    )(agg, cnt, x, wl, wr, b)


def _tc_layer2(agg, cnt, h1, wl, wr, b, wo, bo):
    return pl.pallas_call(
        _tc_layer2_body,
        grid=(_NPAD // _BLK,),
        in_specs=[_agg_spec, _cnt_spec, _row_spec, _w_spec, _w_spec, _b_spec,
                  _w_spec, _b_spec],
        out_specs=(_row_spec, _row_spec),
        out_shape=(jax.ShapeDtypeStruct((_NPAD, _D), jnp.float32),
                   jax.ShapeDtypeStruct((_NPAD, _D), jnp.float32)),
    )(agg, cnt, h1, wl, wr, b, wo, bo)


def kernel(x, edge_index, W1l, b1l, W1r, W2l, b2l, W2r, Wo, bo):
    src = edge_index[0]
    dst = edge_index[1]
    il = jnp.stack([src.reshape(-1, _CH), dst.reshape(-1, _CH)],
                   axis=1).reshape(-1)
    xp = jnp.zeros((_NPAD, _D), jnp.float32).at[:_N].set(x)
    agg1, cnt = _sc_agg(True, xp, il)
    h1 = _tc_layer1(agg1, cnt, xp, W1l, W1r, b1l.reshape(1, _D))
    (agg2,) = _sc_agg(False, h1, il)
    wo_pad = jnp.zeros((_D, _D), jnp.float32).at[:Wo.shape[0]].set(Wo)
    bo_pad = jnp.zeros((1, _D), jnp.float32).at[0, :bo.shape[0]].set(bo)
    h2, logits_pad = _tc_layer2(agg2, cnt, h1, W2l, W2r, b2l.reshape(1, _D),
                                wo_pad, bo_pad)
    return (logits_pad[:_N, :Wo.shape[0]], h2[:_N])
